# baseline (speedup 1.0000x reference)
import functools

import jax
import jax.numpy as jnp
from jax import lax
from jax.experimental import pallas as pl
from jax.experimental.pallas import tpu as pltpu
from jax.experimental.pallas import tpu_sc as plsc

N = 10000
E = 160000
F = 128
FD = 128
CHUNK = 80
NCHUNKS = E // CHUNK
CPAD = 2048
NB = 1000
NBLK = N // NB
WB = 624
AROWS = N + 8
ZPW = 624

_mesh = lambda: plsc.VectorSubcoreMesh(
    core_axis_name="c", subcore_axis_name="s", num_cores=2, num_subcores=16)


def _zero_acc(zeros_hbm, acc, s):
    @pl.when(s < 15)
    def _():
        pltpu.sync_copy(zeros_hbm.at[pl.ds(0, ZPW)],
                        acc.at[pl.ds(s * ZPW, ZPW)])

    @pl.when(s == 15)
    def _():
        pltpu.sync_copy(zeros_hbm.at[pl.ds(0, AROWS - 15 * ZPW)],
                        acc.at[pl.ds(15 * ZPW, AROWS - 15 * ZPW)])


def _writeback_n(acc, out2d, s):
    @pl.when(s < 15)
    def _():
        pltpu.sync_copy(acc.at[pl.ds(s * WB, WB)], out2d.at[pl.ds(s * WB, WB)])

    @pl.when(s == 15)
    def _():
        pltpu.sync_copy(acc.at[pl.ds(15 * WB, N - 15 * WB)],
                        out2d.at[pl.ds(15 * WB, N - 15 * WB)])


def _make_prop(split_edges):
    nrows = 64 if split_edges else 128
    nbuf = 4
    grp = 32
    ngroups = nrows // grp

    @functools.partial(
        pl.kernel,
        out_type=jax.ShapeDtypeStruct((2, N, F), jnp.float32),
        mesh=_mesh(),
        scratch_types=(
            [pltpu.VMEM((grp, CHUNK), jnp.int32),
             pltpu.VMEM((grp, CHUNK), jnp.int32)]
            + [pltpu.VMEM((CHUNK, F), jnp.float32) for _ in range(nbuf)]
            + [pltpu.VMEM_SHARED((AROWS, F), jnp.float32)]
            + [pltpu.SemaphoreType.DMA for _ in range(2 * nbuf)]
        ),
    )
    def prop(src2_hbm, dst2_hbm, xs_hbm, zeros_hbm, out_hbm, *refs):
        gi_blk, di_blk = refs[0], refs[1]
        rows = refs[2:2 + nbuf]
        acc = refs[2 + nbuf]
        sems = refs[3 + nbuf:3 + 2 * nbuf]
        ssems = refs[3 + 2 * nbuf:]
        c = lax.axis_index("c")
        s = lax.axis_index("s")
        _zero_acc(zeros_hbm, acc, s)
        if split_edges:
            base = (c * 16 + s) * nrows
            gsel = 0
        else:
            base = s * nrows
            gsel = c
        plsc.subcore_barrier()

        def start_gather(j, k):
            pltpu.async_copy(xs_hbm.at[gi_blk.at[j]], rows[k], sems[k])

        def wait_gather(j, k):
            pltpu.make_async_copy(xs_hbm.at[gi_blk.at[j]], rows[k],
                                  sems[k]).wait()

        def start_scatter(j, k):
            pltpu.async_copy(rows[k], acc.at[di_blk.at[j]], ssems[k],
                             add=True)

        def wait_scatter(k):
            pltpu.make_async_copy(rows[k], acc.at[di_blk.at[0]],
                                  ssems[k]).wait()

        def group_body(g, carry):
            pltpu.sync_copy(src2_hbm.at[gsel, pl.ds(base + g * grp, grp)],
                            gi_blk)
            pltpu.sync_copy(dst2_hbm.at[pl.ds(base + g * grp, grp)], di_blk)
            for k in range(nbuf - 1):
                start_gather(k, k)
            wait_gather(0, 0)
            start_scatter(0, 0)
            start_gather(nbuf - 1, nbuf - 1)

            def body(i, carry2):
                for m in range(nbuf):
                    j = i * nbuf + m + 1
                    k = (m + 1) % nbuf
                    kp = m % nbuf
                    wait_gather(j, k)
                    start_scatter(j, k)
                    wait_scatter(kp)
                    start_gather(j + nbuf - 1, kp)
                return carry2

            lax.fori_loop(0, (grp - nbuf) // nbuf, body, 0)
            for j in range(grp - nbuf + 1, grp):
                wait_gather(j, j % nbuf)
                start_scatter(j, j % nbuf)
            for k in range(nbuf):
                wait_scatter(k)
            return carry

        lax.fori_loop(0, ngroups, group_body, 0)

        plsc.subcore_barrier()
        _writeback_n(acc, out_hbm.at[c], s)

    return prop


_prop2 = _make_prop(split_edges=False)
_prop1 = _make_prop(split_edges=True)

_DEG_ROWS = 64


@functools.partial(
    pl.kernel,
    out_type=jax.ShapeDtypeStruct((2, N, FD), jnp.float32),
    mesh=_mesh(),
    scratch_types=[
        pltpu.VMEM((_DEG_ROWS, CHUNK), jnp.int32),
        pltpu.VMEM((CHUNK, FD), jnp.float32),
        pltpu.VMEM_SHARED((AROWS, FD), jnp.float32),
        pltpu.SemaphoreType.DMA,
    ],
)
def _deg_kernel(dst2_hbm, ones_hbm, zeros_hbm, out_hbm, di_blk, ones_v, acc,
                sem):
    c = lax.axis_index("c")
    s = lax.axis_index("s")
    _zero_acc(zeros_hbm, acc, s)
    wid = c * 16 + s
    pltpu.sync_copy(dst2_hbm.at[pl.ds(wid * _DEG_ROWS, _DEG_ROWS)], di_blk)
    pltpu.sync_copy(ones_hbm, ones_v)
    plsc.subcore_barrier()

    def body(i, carry):
        pltpu.sync_copy(ones_v, acc.at[di_blk.at[i]], add=True)
        return carry

    lax.fori_loop(0, _DEG_ROWS, body, 0)
    plsc.subcore_barrier()
    _writeback_n(acc, out_hbm.at[c], s)


def _tc_pre_body(x_ref, degp_ref, xs0_ref, dinv16_ref):
    deg = degp_ref[0, :, 0] + degp_ref[1, :, 0] + 1.0
    dinv = lax.rsqrt(deg)
    dinv16_ref[...] = jnp.broadcast_to(dinv[:, None], (NB, 16))
    xs = x_ref[...] * dinv[:, None]
    for j in range(2):
        xs0_ref[j] = xs[:, F * j:F * (j + 1)]


def _tc1a_body(s0_ref, xs0_ref, dinv16_ref, w1_ref, b1_ref, wma_ref,
               h1_ref, xs1a_ref):
    dinv = dinv16_ref[:, 0:1]
    z0 = jnp.concatenate(
        [s0_ref[j] + xs0_ref[j] for j in range(2)], axis=1) * dinv
    h = jnp.maximum(
        jnp.dot(z0, w1_ref[...], preferred_element_type=jnp.float32)
        + b1_ref[...], 0.0)
    h1_ref[...] = h
    ma = jnp.dot(h, wma_ref[...], preferred_element_type=jnp.float32)
    xs1a = ma * dinv
    for j in range(2):
        xs1a_ref[j] = xs1a[:, F * j:F * (j + 1)]


def _tc1b_body(h1_ref, dinv16_ref, wmb_ref, xs1b_ref):
    dinv = dinv16_ref[:, 0:1]
    mb = jnp.dot(h1_ref[...], wmb_ref[...], preferred_element_type=jnp.float32)
    xs1b = mb * dinv
    for j in range(2):
        xs1b_ref[j] = xs1b[:, F * j:F * (j + 1)]


def _tc2a_body(s1a_ref, xs1a_ref, dinv16_ref, bma_ref, w2pa_ref, c2a_ref):
    dinv = dinv16_ref[:, 0:1]
    z1a = (jnp.concatenate([s1a_ref[j] for j in range(2)], axis=1)
           + jnp.concatenate([xs1a_ref[j] for j in range(2)], axis=1)
           ) * dinv + bma_ref[...]
    h2a = jnp.maximum(z1a, 0.0)
    c2a_ref[...] = jnp.dot(h2a, w2pa_ref[...],
                           preferred_element_type=jnp.float32)


def _tc2b_body(s1b_ref, xs1b_ref, dinv16_ref, bmb_ref, w2pb_ref, c2a_ref,
               xs2_ref):
    dinv = dinv16_ref[:, 0:1]
    z1b = (jnp.concatenate([s1b_ref[j] for j in range(2)], axis=1)
           + jnp.concatenate([xs1b_ref[j] for j in range(2)], axis=1)
           ) * dinv + bmb_ref[...]
    h2b = jnp.maximum(z1b, 0.0)
    c2 = c2a_ref[...] + jnp.dot(h2b, w2pb_ref[...],
                                preferred_element_type=jnp.float32)
    xs2_ref[...] = c2 * dinv


def _tc3_body(s2p_ref, xs2_ref, dinv16_ref, b2_ref, out_ref):
    dinv = dinv16_ref[:, 0:1]
    z = (s2p_ref[0] + s2p_ref[1] + xs2_ref[...]) * dinv
    out_ref[...] = z[:, :64] + b2_ref[...]


def _row3(d0, d2):
    return pl.BlockSpec((d0, NB, d2), lambda i: (0, i, 0))


def _row2(d1):
    return pl.BlockSpec((NB, d1), lambda i: (i, 0))


def _full(shape):
    return pl.BlockSpec(shape, lambda i: tuple(0 for _ in shape))


def kernel(x, adj, W1, b1, Wm, bm, W2, b2):
    src32 = adj[0].astype(jnp.int32)
    dst32 = adj[1].astype(jnp.int32)
    npad = CPAD * CHUNK - E
    srcp = jnp.concatenate([src32, jnp.zeros((npad,), jnp.int32)])
    src2 = jnp.stack([srcp, srcp + N]).reshape(2, CPAD, CHUNK)
    dst2 = jnp.concatenate(
        [dst32, jnp.full((npad,), N, jnp.int32)]).reshape(CPAD, CHUNK)
    zerosF = jnp.zeros((AROWS - 15 * ZPW, F), jnp.float32)
    onesD = jnp.ones((CHUNK, FD), jnp.float32)
    W2p = jnp.pad(W2, ((0, 0), (0, F - 64)))

    degp = _deg_kernel(dst2, onesD, zerosF)

    xs0, dinv16 = pl.pallas_call(
        _tc_pre_body,
        grid=(NBLK,),
        in_specs=[_row2(256), _row3(2, FD)],
        out_specs=[_row3(2, F), _row2(16)],
        out_shape=[jax.ShapeDtypeStruct((2, N, F), jnp.float32),
                   jax.ShapeDtypeStruct((N, 16), jnp.float32)],
    )(x, degp)

    s0 = _prop2(src2, dst2, xs0.reshape(2 * N, F), zerosF)

    h1, xs1a = pl.pallas_call(
        _tc1a_body,
        grid=(NBLK,),
        in_specs=[_row3(2, F), _row3(2, F), _row2(16), _full((256, 512)),
                  _full((1, 512)), _full((512, 256))],
        out_specs=[_row2(512), _row3(2, F)],
        out_shape=[jax.ShapeDtypeStruct((N, 512), jnp.float32),
                   jax.ShapeDtypeStruct((2, N, F), jnp.float32)],
    )(s0, xs0, dinv16, W1, b1.reshape(1, 512), Wm[:, :256])

    s1a = _prop2(src2, dst2, xs1a.reshape(2 * N, F), zerosF)

    xs1b = pl.pallas_call(
        _tc1b_body,
        grid=(NBLK,),
        in_specs=[_row2(512), _row2(16), _full((512, 256))],
        out_specs=_row3(2, F),
        out_shape=jax.ShapeDtypeStruct((2, N, F), jnp.float32),
    )(h1, dinv16, Wm[:, 256:])

    s1b = _prop2(src2, dst2, xs1b.reshape(2 * N, F), zerosF)

    c2a = pl.pallas_call(
        _tc2a_body,
        grid=(NBLK,),
        in_specs=[_row3(2, F), _row3(2, F), _row2(16), _full((1, 256)),
                  _full((256, F))],
        out_specs=_row2(F),
        out_shape=jax.ShapeDtypeStruct((N, F), jnp.float32),
    )(s1a, xs1a, dinv16, bm[:256].reshape(1, 256), W2p[:256])

    xs2 = pl.pallas_call(
        _tc2b_body,
        grid=(NBLK,),
        in_specs=[_row3(2, F), _row3(2, F), _row2(16), _full((1, 256)),
                  _full((256, F)), _row2(F)],
        out_specs=_row2(F),
        out_shape=jax.ShapeDtypeStruct((N, F), jnp.float32),
    )(s1b, xs1b, dinv16, bm[256:].reshape(1, 256), W2p[256:], c2a)

    s2p = _prop1(src2, dst2, xs2, zerosF)

    out = pl.pallas_call(
        _tc3_body,
        grid=(NBLK,),
        in_specs=[_row3(2, F), _row2(F), _row2(16), _full((1, 64))],
        out_specs=_row2(64),
        out_shape=jax.ShapeDtypeStruct((N, 64), jnp.float32),
    )(s2p, xs2, dinv16, b2.reshape(1, 64))

    return out


# speedup vs baseline: 1.0693x; 1.0693x over previous
import functools

import jax
import jax.numpy as jnp
from jax import lax
from jax.experimental import pallas as pl
from jax.experimental.pallas import tpu as pltpu
from jax.experimental.pallas import tpu_sc as plsc

N = 10000
E = 160000
F = 128
FD = 16
CHUNK = 80
NCHUNKS = E // CHUNK
NB = 1000
NBLK = N // NB
WB = 624

_mesh = lambda: plsc.VectorSubcoreMesh(
    core_axis_name="c", subcore_axis_name="s", num_cores=2, num_subcores=16)


def _copy_n_rows(src_of, dst_of, s):
    @pl.when(s < 15)
    def _():
        pltpu.sync_copy(src_of(s * WB, WB), dst_of(s * WB, WB))

    @pl.when(s == 15)
    def _():
        pltpu.sync_copy(src_of(15 * WB, N - 15 * WB),
                        dst_of(15 * WB, N - 15 * WB))


def _make_prop(split_edges):
    @functools.partial(
        pl.kernel,
        out_type=jax.ShapeDtypeStruct((2, N, F), jnp.float32),
        mesh=_mesh(),
        scratch_types=[
            pltpu.VMEM((CHUNK,), jnp.int32),
            pltpu.VMEM((CHUNK,), jnp.int32),
            pltpu.VMEM((CHUNK,), jnp.int32),
            pltpu.VMEM((CHUNK, F), jnp.float32),
            pltpu.VMEM_SHARED((N, F), jnp.float32),
            pltpu.SemaphoreType.DMA,
        ],
    )
    def prop(xs_hbm, src_hbm, dst_hbm, zeros_hbm, out_hbm,
             srcv, dstv, gi, rows, acc, sem):
        c = lax.axis_index("c")
        s = lax.axis_index("s")
        _copy_n_rows(lambda o, n: zeros_hbm.at[pl.ds(0, n)],
                     lambda o, n: acc.at[pl.ds(o, n)], s)
        plsc.subcore_barrier()

        def do_chunk(chunk_id, goff):
            base = chunk_id * CHUNK
            pltpu.sync_copy(src_hbm.at[pl.ds(base, CHUNK)], srcv)
            pltpu.sync_copy(dst_hbm.at[pl.ds(base, CHUNK)], dstv)
            if goff is None:
                pltpu.async_copy(xs_hbm.at[srcv], rows, sem).wait()
            else:
                for j in range(CHUNK // 16):
                    sl = pl.ds(16 * j, 16)
                    gi[sl] = srcv[sl] + goff
                pltpu.async_copy(xs_hbm.at[gi], rows, sem).wait()
            pltpu.sync_copy(rows, acc.at[dstv], add=True)

        if not split_edges:
            goff = c * N

            def body(i, carry):
                do_chunk(s + 16 * i, goff)
                return carry

            lax.fori_loop(0, NCHUNKS // 16, body, 0)
        else:
            wid = c * 16 + s

            def body(i, carry):
                do_chunk(wid + 32 * i, None)
                return carry

            lax.fori_loop(0, NCHUNKS // 32, body, 0)

            @pl.when(wid < NCHUNKS - 32 * (NCHUNKS // 32))
            def _():
                do_chunk(32 * (NCHUNKS // 32) + wid, None)

        plsc.subcore_barrier()
        _copy_n_rows(lambda o, n: acc.at[pl.ds(o, n)],
                     lambda o, n: out_hbm.at[c, pl.ds(o, n)], s)

    return prop


_prop2 = _make_prop(split_edges=False)
_prop1 = _make_prop(split_edges=True)


@functools.partial(
    pl.kernel,
    out_type=jax.ShapeDtypeStruct((2, N, FD), jnp.float32),
    mesh=_mesh(),
    scratch_types=[
        pltpu.VMEM((CHUNK,), jnp.int32),
        pltpu.VMEM((CHUNK, FD), jnp.float32),
        pltpu.VMEM_SHARED((N, FD), jnp.float32),
    ],
)
def _deg_kernel(dst_hbm, ones_hbm, zeros_hbm, out_hbm, dstv, rows, acc):
    c = lax.axis_index("c")
    s = lax.axis_index("s")
    pltpu.sync_copy(ones_hbm, rows)
    _copy_n_rows(lambda o, n: zeros_hbm.at[pl.ds(0, n)],
                 lambda o, n: acc.at[pl.ds(o, n)], s)
    plsc.subcore_barrier()
    wid = c * 16 + s

    def do_chunk(chunk_id):
        pltpu.sync_copy(dst_hbm.at[pl.ds(chunk_id * CHUNK, CHUNK)], dstv)
        pltpu.sync_copy(rows, acc.at[dstv], add=True)

    def body(i, carry):
        do_chunk(wid + 32 * i)
        return carry

    lax.fori_loop(0, NCHUNKS // 32, body, 0)

    @pl.when(wid < NCHUNKS - 32 * (NCHUNKS // 32))
    def _():
        do_chunk(32 * (NCHUNKS // 32) + wid)

    plsc.subcore_barrier()
    _copy_n_rows(lambda o, n: acc.at[pl.ds(o, n)],
                 lambda o, n: out_hbm.at[c, pl.ds(o, n)], s)


def _tc_pre_body(x_ref, degp_ref, xs0_ref, dinv16_ref):
    deg = degp_ref[0, :, 0] + degp_ref[1, :, 0] + 1.0
    dinv = lax.rsqrt(deg)
    dinv16_ref[...] = jnp.broadcast_to(dinv[:, None], (NB, 16))
    xs = x_ref[...] * dinv[:, None]
    for j in range(2):
        xs0_ref[j] = xs[:, F * j:F * (j + 1)]


def _tc1_body(s0_ref, xs0_ref, dinv16_ref, w1_ref, b1_ref, wm_ref,
              xs1a_ref, xs1b_ref):
    dinv = dinv16_ref[:, 0:1]
    z0 = jnp.concatenate(
        [s0_ref[j] + xs0_ref[j] for j in range(2)], axis=1) * dinv
    h = jnp.maximum(
        jnp.dot(z0, w1_ref[...], preferred_element_type=jnp.float32)
        + b1_ref[...], 0.0)
    m = jnp.dot(h, wm_ref[...], preferred_element_type=jnp.float32)
    xs1 = m * dinv
    for j in range(2):
        xs1a_ref[j] = xs1[:, F * j:F * (j + 1)]
        xs1b_ref[j] = xs1[:, 256 + F * j:256 + F * (j + 1)]


def _tc2_body(s1a_ref, s1b_ref, xs1a_ref, xs1b_ref, dinv16_ref, bm_ref,
              w2p_ref, xs2_ref):
    dinv = dinv16_ref[:, 0:1]
    scat = jnp.concatenate(
        [s1a_ref[j] for j in range(2)] + [s1b_ref[j] for j in range(2)],
        axis=1)
    xcat = jnp.concatenate(
        [xs1a_ref[j] for j in range(2)] + [xs1b_ref[j] for j in range(2)],
        axis=1)
    z1 = (scat + xcat) * dinv + bm_ref[...]
    h2 = jnp.maximum(z1, 0.0)
    c2 = jnp.dot(h2, w2p_ref[...], preferred_element_type=jnp.float32)
    xs2_ref[...] = c2 * dinv


def _tc3_body(s2p_ref, xs2_ref, dinv16_ref, b2_ref, out_ref):
    dinv = dinv16_ref[:, 0:1]
    z = (s2p_ref[0] + s2p_ref[1] + xs2_ref[...]) * dinv
    out_ref[...] = z[:, :64] + b2_ref[...]


def _row3(d0, d2):
    return pl.BlockSpec((d0, NB, d2), lambda i: (0, i, 0))


def _row2(d1):
    return pl.BlockSpec((NB, d1), lambda i: (i, 0))


def _full(shape):
    return pl.BlockSpec(shape, lambda i: tuple(0 for _ in shape))


def kernel(x, adj, W1, b1, Wm, bm, W2, b2):
    src32 = adj[0].astype(jnp.int32)
    dst32 = adj[1].astype(jnp.int32)
    zerosF = jnp.zeros((N - 15 * WB, F), jnp.float32)
    zerosD = jnp.zeros((N - 15 * WB, FD), jnp.float32)
    onesD = jnp.ones((CHUNK, FD), jnp.float32)
    W2p = jnp.pad(W2, ((0, 0), (0, F - 64)))

    degp = _deg_kernel(dst32, onesD, zerosD)

    xs0, dinv16 = pl.pallas_call(
        _tc_pre_body,
        grid=(NBLK,),
        in_specs=[_row2(256), _row3(2, FD)],
        out_specs=[_row3(2, F), _row2(16)],
        out_shape=[jax.ShapeDtypeStruct((2, N, F), jnp.float32),
                   jax.ShapeDtypeStruct((N, 16), jnp.float32)],
    )(x, degp)

    s0 = _prop2(xs0.reshape(2 * N, F), src32, dst32, zerosF)

    xs1a, xs1b = pl.pallas_call(
        _tc1_body,
        grid=(NBLK,),
        in_specs=[_row3(2, F), _row3(2, F), _row2(16), _full((256, 512)),
                  _full((1, 512)), _full((512, 512))],
        out_specs=[_row3(2, F), _row3(2, F)],
        out_shape=[jax.ShapeDtypeStruct((2, N, F), jnp.float32),
                   jax.ShapeDtypeStruct((2, N, F), jnp.float32)],
    )(s0, xs0, dinv16, W1, b1.reshape(1, 512), Wm)

    s1a = _prop2(xs1a.reshape(2 * N, F), src32, dst32, zerosF)
    s1b = _prop2(xs1b.reshape(2 * N, F), src32, dst32, zerosF)

    xs2 = pl.pallas_call(
        _tc2_body,
        grid=(NBLK,),
        in_specs=[_row3(2, F), _row3(2, F), _row3(2, F), _row3(2, F),
                  _row2(16), _full((1, 512)), _full((512, F))],
        out_specs=_row2(F),
        out_shape=jax.ShapeDtypeStruct((N, F), jnp.float32),
    )(s1a, s1b, xs1a, xs1b, dinv16, bm.reshape(1, 512), W2p)

    s2p = _prop1(xs2, src32, dst32, zerosF)

    out = pl.pallas_call(
        _tc3_body,
        grid=(NBLK,),
        in_specs=[_row3(2, F), _row2(F), _row2(16), _full((1, 64))],
        out_specs=_row2(64),
        out_shape=jax.ShapeDtypeStruct((N, 64), jnp.float32),
    )(s2p, xs2, dinv16, b2.reshape(1, 64))

    return out
